# Initial kernel scaffold; baseline (speedup 1.0000x reference)
import jax
import jax.numpy as jnp
from jax.experimental import pallas as pl


def kernel(emb, fc_w, fc_b, g_edge_index, neg_edge_index):
    raise NotImplementedError("write your pallas kernel here")


# speedup vs baseline: 1.1683x; 1.1683x over previous
import functools

import jax
import jax.numpy as jnp
from jax import lax
from jax.experimental import pallas as pl
from jax.experimental.pallas import tpu as pltpu
from jax.experimental.pallas import tpu_sc as plsc

N = 10000
NPAD = 10240
D = 128
E = 320000

NC = 2
NS = 16
NW = NC * NS

DEG_W = 16
HALF = NPAD // NC
HROWS = HALF // NS
ACC_R = HALF + 8
DUMP = HALF

EPS = E // NS
GB = 80
GCH = EPS // GB
SCB = 16
NSUB = GB // SCB
SCCH = EPS // SCB

SB = 80
SCH = (2 * E) // NW // SB

_MESH = plsc.VectorSubcoreMesh(
    core_axis_name="c", subcore_axis_name="s", num_cores=NC, num_subcores=NS
)


def _dedup_vec(dv, lane):
    fp = lane
    for p in range(SCB - 2, -1, -1):
        match = jnp.logical_and(dv == dv[p], lane > p)
        fp = jnp.where(match, p, fp)
    return fp


@functools.partial(
    pl.kernel,
    out_type=jax.ShapeDtypeStruct((NPAD, D), jnp.float32),
    mesh=_MESH,
    scratch_types=[
        pltpu.VMEM((GCH, GB), jnp.int32),
        pltpu.VMEM((GCH, GB), jnp.int32),
        pltpu.VMEM((NSUB, SCB), jnp.int32),
        pltpu.VMEM((GB, D), jnp.float32),
        pltpu.VMEM_SHARED((ACC_R, D), jnp.float32),
        pltpu.SemaphoreType.DMA,
    ],
)
def _prop_kernel(x_hbm, src3_hbm, dst3_hbm, zeros_hbm, out_hbm,
                 sidx, didx, dsx, rows, acc, sem):
    c = lax.axis_index("c")
    s = lax.axis_index("s")
    base = c * HALF
    pltpu.sync_copy(zeros_hbm.at[pl.ds(s * HROWS, HROWS)],
                    acc.at[pl.ds(s * HROWS, HROWS)])
    pltpu.sync_copy(src3_hbm.at[s], sidx)
    pltpu.sync_copy(dst3_hbm.at[s], didx)
    lane = lax.iota(jnp.int32, 16)
    plsc.subcore_barrier()

    def chunk(j, carry):
        pltpu.async_copy(x_hbm.at[sidx.at[j]], rows, sem).wait()
        for m in range(NSUB):
            dv = didx[j, pl.ds(m * SCB, SCB)]
            fp = _dedup_vec(dv, lane)
            for e in range(1, SCB):
                fpe = fp[e]

                @pl.when(fpe < e)
                def _():
                    for k in range(D // 16):
                        rows[m * SCB + fpe, pl.ds(k * 16, 16)] = (
                            rows[m * SCB + fpe, pl.ds(k * 16, 16)]
                            + rows[m * SCB + e, pl.ds(k * 16, 16)]
                        )

            ldv = dv - base
            keep = jnp.logical_and(
                jnp.logical_and(ldv >= 0, ldv < HALF), fp == lane)
            dsx[m] = jnp.where(keep, ldv, DUMP)
            pltpu.sync_copy(rows.at[pl.ds(m * SCB, SCB)],
                            acc.at[dsx.at[m]], add=True)
        return carry

    lax.fori_loop(0, GCH, chunk, 0)
    plsc.subcore_barrier()
    pltpu.sync_copy(acc.at[pl.ds(s * HROWS, HROWS)],
                    out_hbm.at[pl.ds(c * HALF + s * HROWS, HROWS)])


@functools.partial(
    pl.kernel,
    out_type=jax.ShapeDtypeStruct((NW, SCH, SB, 16), jnp.float32),
    mesh=_MESH,
    scratch_types=[
        pltpu.VMEM((SCH, SB), jnp.int32),
        pltpu.VMEM((SCH, SB), jnp.int32),
        pltpu.VMEM((SB, D), jnp.float32),
        pltpu.VMEM((SB, D), jnp.float32),
        pltpu.VMEM((SB, 16), jnp.float32),
        pltpu.SemaphoreType.DMA,
        pltpu.SemaphoreType.DMA,
    ],
)
def _score_kernel(h_hbm, uix_hbm, vix_hbm, out_hbm, uix, vix, ub, vb, part, semu, semv):
    c = lax.axis_index("c")
    s = lax.axis_index("s")
    wid = s * NC + c
    pltpu.sync_copy(uix_hbm.at[wid], uix)
    pltpu.sync_copy(vix_hbm.at[wid], vix)

    def chunk(j, carry):
        cu = pltpu.async_copy(h_hbm.at[uix.at[j]], ub, semu)
        cv = pltpu.async_copy(h_hbm.at[vix.at[j]], vb, semv)
        cu.wait()
        cv.wait()

        def edge(e, c2):
            acc = ub[e, pl.ds(0, 16)] * vb[e, pl.ds(0, 16)]
            for k in range(1, D // 16):
                acc = acc + ub[e, pl.ds(k * 16, 16)] * vb[e, pl.ds(k * 16, 16)]
            part[e] = acc
            return c2

        lax.fori_loop(0, SB, edge, 0)
        pltpu.sync_copy(part, out_hbm.at[wid, j])
        return carry

    lax.fori_loop(0, SCH, chunk, 0)


RB = 1024


def _scale1_body(d_ref, emb_ref, out_ref):
    d = jnp.maximum(d_ref[...][:, :1], 1.0)
    out_ref[...] = emb_ref[...] * lax.rsqrt(d)


def _tc_scale1(deg, emb_p):
    return pl.pallas_call(
        _scale1_body,
        out_shape=jax.ShapeDtypeStruct((NPAD, D), jnp.float32),
        grid=(NPAD // RB,),
        in_specs=[
            pl.BlockSpec((RB, D), lambda i: (i, 0)),
            pl.BlockSpec((RB, D), lambda i: (i, 0)),
        ],
        out_specs=pl.BlockSpec((RB, D), lambda i: (i, 0)),
    )(deg, emb_p)


def _scale2_body(d_ref, p_ref, out_ref):
    d = jnp.maximum(d_ref[...][:, :1], 1.0)
    out_ref[...] = p_ref[...] / d


def _tc_scale2(deg, p):
    return pl.pallas_call(
        _scale2_body,
        out_shape=jax.ShapeDtypeStruct((NPAD, D), jnp.float32),
        grid=(NPAD // RB,),
        in_specs=[
            pl.BlockSpec((RB, D), lambda i: (i, 0)),
            pl.BlockSpec((RB, D), lambda i: (i, 0)),
        ],
        out_specs=pl.BlockSpec((RB, D), lambda i: (i, 0)),
    )(deg, p)


def _final_body(d_ref, p_ref, w_ref, b_ref, out_ref):
    d = jnp.maximum(d_ref[...][:, :1], 1.0)
    x = p_ref[...] * lax.rsqrt(d)
    out_ref[...] = (
        lax.dot_general(
            x, w_ref[...], (((1,), (1,)), ((), ())),
            preferred_element_type=jnp.float32,
            precision=lax.Precision.HIGHEST,
        )
        + b_ref[...]
    )


def _tc_final(deg, p, fc_w, fc_b2):
    return pl.pallas_call(
        _final_body,
        out_shape=jax.ShapeDtypeStruct((NPAD, D), jnp.float32),
        grid=(NPAD // RB,),
        in_specs=[
            pl.BlockSpec((RB, D), lambda i: (i, 0)),
            pl.BlockSpec((RB, D), lambda i: (i, 0)),
            pl.BlockSpec((D, D), lambda i: (0, 0)),
            pl.BlockSpec((1, D), lambda i: (0, 0)),
        ],
        out_specs=pl.BlockSpec((RB, D), lambda i: (i, 0)),
    )(deg, p, fc_w, fc_b2)


RB2 = 4000


def _rowsum_body(x_ref, out_ref):
    r16 = lax.broadcasted_iota(jnp.int32, (D, 8), 0) // 16
    cix = lax.broadcasted_iota(jnp.int32, (D, 8), 1)
    m = (r16 == cix).astype(jnp.float32)
    out_ref[...] = lax.dot_general(
        x_ref[...], m, (((1,), (0,)), ((), ())),
        preferred_element_type=jnp.float32,
        precision=lax.Precision.HIGHEST,
    )


def _tc_rowsum(scores16):
    rows = scores16.shape[0]
    return pl.pallas_call(
        _rowsum_body,
        out_shape=jax.ShapeDtypeStruct((rows, 8), jnp.float32),
        grid=(rows // RB2,),
        in_specs=[pl.BlockSpec((RB2, D), lambda i: (i, 0))],
        out_specs=pl.BlockSpec((RB2, 8), lambda i: (i, 0)),
    )(scores16)


def kernel(emb, fc_w, fc_b, g_edge_index, neg_edge_index):
    src3 = g_edge_index[0].reshape(NS, GCH, GB)
    dst3 = g_edge_index[1].reshape(NS, GCH, GB)
    emb_p = jnp.pad(emb, ((0, NPAD - N), (0, 0)))
    zeros_nd = jnp.zeros((NPAD, D), jnp.float32)
    ones_nd = jnp.ones((NPAD, D), jnp.float32)
    deg = _prop_kernel(ones_nd, src3, dst3, zeros_nd)
    t1 = _tc_scale1(deg, emb_p)
    p1 = _prop_kernel(t1, src3, dst3, zeros_nd)
    t3 = _tc_scale2(deg, p1)
    p2 = _prop_kernel(t3, src3, dst3, zeros_nd)
    h = _tc_final(deg, p2, fc_w, fc_b.reshape(1, D))
    u = jnp.concatenate([g_edge_index[0], neg_edge_index[0]]).reshape(NW, SCH, SB)
    v = jnp.concatenate([g_edge_index[1], neg_edge_index[1]]).reshape(NW, SCH, SB)
    scores16 = _score_kernel(h, u, v).reshape(2 * E * 16 // D, D)
    scores = _tc_rowsum(scores16).reshape(-1)
    return (scores[:E].reshape(E, 1), scores[E:].reshape(E, 1))


# speedup vs baseline: 1.3809x; 1.1820x over previous
import functools

import jax
import jax.numpy as jnp
from jax import lax
from jax.experimental import pallas as pl
from jax.experimental.pallas import tpu as pltpu
from jax.experimental.pallas import tpu_sc as plsc

N = 10000
NPAD = 10240
D = 128
E = 320000

NC = 2
NS = 16
NW = NC * NS

DEG_W = 16
HALF = NPAD // NC
HROWS = HALF // NS
ACC_R = HALF + 8
DUMP = HALF

EPS = E // NS
GB = 80
GCH = EPS // GB
SCB = 16
NSUB = GB // SCB
SCCH = EPS // SCB

SB = 80
SCH = (2 * E) // NW // SB

_MESH = plsc.VectorSubcoreMesh(
    core_axis_name="c", subcore_axis_name="s", num_cores=NC, num_subcores=NS
)


def _dedup_vec(dv, lane):
    fp = lane
    for p in range(SCB - 2, -1, -1):
        match = jnp.logical_and(dv == dv[p], lane > p)
        fp = jnp.where(match, p, fp)
    return fp


@functools.partial(
    pl.kernel,
    out_type=jax.ShapeDtypeStruct((NPAD, D), jnp.float32),
    mesh=_MESH,
    scratch_types=[
        pltpu.VMEM((GCH, GB), jnp.int32),
        pltpu.VMEM((GCH, GB), jnp.int32),
        pltpu.VMEM((2, NSUB, SCB), jnp.int32),
        pltpu.VMEM((GB, D), jnp.float32),
        pltpu.VMEM((GB, D), jnp.float32),
        pltpu.VMEM_SHARED((ACC_R, D), jnp.float32),
        pltpu.SemaphoreType.DMA,
        pltpu.SemaphoreType.DMA,
    ],
)
def _prop_kernel(x_hbm, src3_hbm, dst3_hbm, zeros_hbm, out_hbm,
                 sidx, didx, dsx, rowsA, rowsB, acc, semA, semB):
    c = lax.axis_index("c")
    s = lax.axis_index("s")
    base = c * HALF
    pltpu.sync_copy(zeros_hbm.at[pl.ds(s * HROWS, HROWS)],
                    acc.at[pl.ds(s * HROWS, HROWS)])
    pltpu.sync_copy(src3_hbm.at[s], sidx)
    pltpu.sync_copy(dst3_hbm.at[s], didx)
    lane = lax.iota(jnp.int32, 16)
    plsc.subcore_barrier()

    def process(j, rows, b):
        for m in range(NSUB):
            dv = didx[j, pl.ds(m * SCB, SCB)]
            fp = _dedup_vec(dv, lane)
            for e in range(1, SCB):
                fpe = fp[e]

                @pl.when(fpe < e)
                def _():
                    for k in range(D // 16):
                        rows[m * SCB + fpe, pl.ds(k * 16, 16)] = (
                            rows[m * SCB + fpe, pl.ds(k * 16, 16)]
                            + rows[m * SCB + e, pl.ds(k * 16, 16)]
                        )

            ldv = dv - base
            keep = jnp.logical_and(
                jnp.logical_and(ldv >= 0, ldv < HALF), fp == lane)
            dsx[b, m] = jnp.where(keep, ldv, DUMP)
            pltpu.sync_copy(rows.at[pl.ds(m * SCB, SCB)],
                            acc.at[dsx.at[b, m]], add=True)

    pltpu.async_copy(x_hbm.at[sidx.at[0]], rowsA, semA)

    def pair(jj, carry):
        j0 = 2 * jj
        pltpu.async_copy(x_hbm.at[sidx.at[j0 + 1]], rowsB, semB)
        pltpu.make_async_copy(x_hbm.at[sidx.at[j0]], rowsA, semA).wait()
        process(j0, rowsA, 0)

        @pl.when(j0 + 2 < GCH)
        def _():
            pltpu.async_copy(x_hbm.at[sidx.at[j0 + 2]], rowsA, semA)

        pltpu.make_async_copy(x_hbm.at[sidx.at[j0 + 1]], rowsB, semB).wait()
        process(j0 + 1, rowsB, 1)
        return carry

    lax.fori_loop(0, GCH // 2, pair, 0)
    plsc.subcore_barrier()
    pltpu.sync_copy(acc.at[pl.ds(s * HROWS, HROWS)],
                    out_hbm.at[pl.ds(c * HALF + s * HROWS, HROWS)])


@functools.partial(
    pl.kernel,
    out_type=jax.ShapeDtypeStruct((NW, SCH, SB, 16), jnp.float32),
    mesh=_MESH,
    scratch_types=[
        pltpu.VMEM((SCH, SB), jnp.int32),
        pltpu.VMEM((SCH, SB), jnp.int32),
        pltpu.VMEM((2, SB, D), jnp.float32),
        pltpu.VMEM((2, SB, D), jnp.float32),
        pltpu.VMEM((SB, 16), jnp.float32),
        pltpu.SemaphoreType.DMA,
        pltpu.SemaphoreType.DMA,
        pltpu.SemaphoreType.DMA,
        pltpu.SemaphoreType.DMA,
    ],
)
def _score_kernel(h_hbm, uix_hbm, vix_hbm, out_hbm, uix, vix, ub2, vb2, part,
                  semuA, semvA, semuB, semvB):
    c = lax.axis_index("c")
    s = lax.axis_index("s")
    wid = s * NC + c
    pltpu.sync_copy(uix_hbm.at[wid], uix)
    pltpu.sync_copy(vix_hbm.at[wid], vix)

    def compute(j, b):
        def edge(e, c2):
            acc = ub2[b, e, pl.ds(0, 16)] * vb2[b, e, pl.ds(0, 16)]
            for k in range(1, D // 16):
                acc = acc + (ub2[b, e, pl.ds(k * 16, 16)]
                             * vb2[b, e, pl.ds(k * 16, 16)])
            part[e] = acc
            return c2

        lax.fori_loop(0, SB, edge, 0)
        pltpu.sync_copy(part, out_hbm.at[wid, j])

    def fetch(j, b, su, sv):
        pltpu.async_copy(h_hbm.at[uix.at[j]], ub2.at[b], su)
        pltpu.async_copy(h_hbm.at[vix.at[j]], vb2.at[b], sv)

    def drain(j, b, su, sv):
        pltpu.make_async_copy(h_hbm.at[uix.at[j]], ub2.at[b], su).wait()
        pltpu.make_async_copy(h_hbm.at[vix.at[j]], vb2.at[b], sv).wait()

    fetch(0, 0, semuA, semvA)

    def pair(jj, carry):
        j0 = 2 * jj
        fetch(j0 + 1, 1, semuB, semvB)
        drain(j0, 0, semuA, semvA)
        compute(j0, 0)

        @pl.when(j0 + 2 < SCH)
        def _():
            fetch(j0 + 2, 0, semuA, semvA)

        drain(j0 + 1, 1, semuB, semvB)
        compute(j0 + 1, 1)
        return carry

    lax.fori_loop(0, SCH // 2, pair, 0)


RB = 1024


def _scale1_body(d_ref, emb_ref, out_ref):
    d = jnp.maximum(d_ref[...][:, :1], 1.0)
    out_ref[...] = emb_ref[...] * lax.rsqrt(d)


def _tc_scale1(deg, emb_p):
    return pl.pallas_call(
        _scale1_body,
        out_shape=jax.ShapeDtypeStruct((NPAD, D), jnp.float32),
        grid=(NPAD // RB,),
        in_specs=[
            pl.BlockSpec((RB, D), lambda i: (i, 0)),
            pl.BlockSpec((RB, D), lambda i: (i, 0)),
        ],
        out_specs=pl.BlockSpec((RB, D), lambda i: (i, 0)),
    )(deg, emb_p)


def _scale2_body(d_ref, p_ref, out_ref):
    d = jnp.maximum(d_ref[...][:, :1], 1.0)
    out_ref[...] = p_ref[...] / d


def _tc_scale2(deg, p):
    return pl.pallas_call(
        _scale2_body,
        out_shape=jax.ShapeDtypeStruct((NPAD, D), jnp.float32),
        grid=(NPAD // RB,),
        in_specs=[
            pl.BlockSpec((RB, D), lambda i: (i, 0)),
            pl.BlockSpec((RB, D), lambda i: (i, 0)),
        ],
        out_specs=pl.BlockSpec((RB, D), lambda i: (i, 0)),
    )(deg, p)


def _final_body(d_ref, p_ref, w_ref, b_ref, out_ref):
    d = jnp.maximum(d_ref[...][:, :1], 1.0)
    x = p_ref[...] * lax.rsqrt(d)
    out_ref[...] = (
        lax.dot_general(
            x, w_ref[...], (((1,), (1,)), ((), ())),
            preferred_element_type=jnp.float32,
            precision=lax.Precision.HIGHEST,
        )
        + b_ref[...]
    )


def _tc_final(deg, p, fc_w, fc_b2):
    return pl.pallas_call(
        _final_body,
        out_shape=jax.ShapeDtypeStruct((NPAD, D), jnp.float32),
        grid=(NPAD // RB,),
        in_specs=[
            pl.BlockSpec((RB, D), lambda i: (i, 0)),
            pl.BlockSpec((RB, D), lambda i: (i, 0)),
            pl.BlockSpec((D, D), lambda i: (0, 0)),
            pl.BlockSpec((1, D), lambda i: (0, 0)),
        ],
        out_specs=pl.BlockSpec((RB, D), lambda i: (i, 0)),
    )(deg, p, fc_w, fc_b2)


RB2 = 4000


def _rowsum_body(x_ref, out_ref):
    r16 = lax.broadcasted_iota(jnp.int32, (D, 8), 0) // 16
    cix = lax.broadcasted_iota(jnp.int32, (D, 8), 1)
    m = (r16 == cix).astype(jnp.float32)
    out_ref[...] = lax.dot_general(
        x_ref[...], m, (((1,), (0,)), ((), ())),
        preferred_element_type=jnp.float32,
        precision=lax.Precision.HIGHEST,
    )


def _tc_rowsum(scores16):
    rows = scores16.shape[0]
    return pl.pallas_call(
        _rowsum_body,
        out_shape=jax.ShapeDtypeStruct((rows, 8), jnp.float32),
        grid=(rows // RB2,),
        in_specs=[pl.BlockSpec((RB2, D), lambda i: (i, 0))],
        out_specs=pl.BlockSpec((RB2, 8), lambda i: (i, 0)),
    )(scores16)


def kernel(emb, fc_w, fc_b, g_edge_index, neg_edge_index):
    src3 = g_edge_index[0].reshape(NS, GCH, GB)
    dst3 = g_edge_index[1].reshape(NS, GCH, GB)
    emb_p = jnp.pad(emb, ((0, NPAD - N), (0, 0)))
    zeros_nd = jnp.zeros((NPAD, D), jnp.float32)
    ones_nd = jnp.ones((NPAD, D), jnp.float32)
    deg = _prop_kernel(ones_nd, src3, dst3, zeros_nd)
    t1 = _tc_scale1(deg, emb_p)
    p1 = _prop_kernel(t1, src3, dst3, zeros_nd)
    t3 = _tc_scale2(deg, p1)
    p2 = _prop_kernel(t3, src3, dst3, zeros_nd)
    h = _tc_final(deg, p2, fc_w, fc_b.reshape(1, D))
    u = jnp.concatenate([g_edge_index[0], neg_edge_index[0]]).reshape(NW, SCH, SB)
    v = jnp.concatenate([g_edge_index[1], neg_edge_index[1]]).reshape(NW, SCH, SB)
    scores16 = _score_kernel(h, u, v).reshape(2 * E * 16 // D, D)
    scores = _tc_rowsum(scores16).reshape(-1)
    return (scores[:E].reshape(E, 1), scores[E:].reshape(E, 1))


# speedup vs baseline: 1.4666x; 1.0620x over previous
import functools

import jax
import jax.numpy as jnp
from jax import lax
from jax.experimental import pallas as pl
from jax.experimental.pallas import tpu as pltpu
from jax.experimental.pallas import tpu_sc as plsc

N = 10000
NPAD = 10240
D = 128
E = 320000

NC = 2
NS = 16
NW = NC * NS

DEG_W = 16
HALF = NPAD // NC
HROWS = HALF // NS
ACC_R = HALF + 8
DUMP = HALF

EPS = E // NS
GB = 80
GCH = EPS // GB
SCB = 16
NSUB = GB // SCB
SCCH = EPS // SCB

SB = 80
SCH = (2 * E) // NW // SB

_MESH = plsc.VectorSubcoreMesh(
    core_axis_name="c", subcore_axis_name="s", num_cores=NC, num_subcores=NS
)


def _dedup_vec(dv, lane):
    fp = lane
    for p in range(SCB - 2, -1, -1):
        match = jnp.logical_and(dv == dv[p], lane > p)
        fp = jnp.where(match, p, fp)
    return fp


@functools.partial(
    pl.kernel,
    out_type=jax.ShapeDtypeStruct((NPAD, D), jnp.float32),
    mesh=_MESH,
    scratch_types=[
        pltpu.VMEM((GCH, GB), jnp.int32),
        pltpu.VMEM((GCH, GB), jnp.int32),
        pltpu.VMEM((2, NSUB, SCB), jnp.int32),
        pltpu.VMEM((GB, D), jnp.float32),
        pltpu.VMEM((GB, D), jnp.float32),
        pltpu.VMEM_SHARED((ACC_R, D), jnp.float32),
        pltpu.SemaphoreType.DMA,
        pltpu.SemaphoreType.DMA,
        pltpu.SemaphoreType.DMA,
    ],
)
def _prop_kernel(x_hbm, src3_hbm, dst3_hbm, zeros_hbm, out_hbm,
                 sidx, didx, dsx, rowsA, rowsB, acc, semA, semB, semS):
    c = lax.axis_index("c")
    s = lax.axis_index("s")
    base = c * HALF
    pltpu.sync_copy(zeros_hbm.at[pl.ds(s * HROWS, HROWS)],
                    acc.at[pl.ds(s * HROWS, HROWS)])
    pltpu.sync_copy(src3_hbm.at[s], sidx)
    pltpu.sync_copy(dst3_hbm.at[s], didx)
    lane = lax.iota(jnp.int32, 16)
    plsc.subcore_barrier()

    def process(j, rows, b):
        descs = []
        for m in range(NSUB):
            dv = didx[j, pl.ds(m * SCB, SCB)]
            fp = _dedup_vec(dv, lane)
            for e in range(1, SCB):
                fpe = fp[e]

                @pl.when(fpe < e)
                def _():
                    for k in range(D // 16):
                        rows[m * SCB + fpe, pl.ds(k * 16, 16)] = (
                            rows[m * SCB + fpe, pl.ds(k * 16, 16)]
                            + rows[m * SCB + e, pl.ds(k * 16, 16)]
                        )

            ldv = dv - base
            keep = jnp.logical_and(
                jnp.logical_and(ldv >= 0, ldv < HALF), fp == lane)
            dsx[b, m] = jnp.where(keep, ldv, DUMP)
            descs.append(pltpu.async_copy(
                rows.at[pl.ds(m * SCB, SCB)],
                acc.at[dsx.at[b, m]], semS, add=True))
        for dd in descs:
            dd.wait()

    pltpu.async_copy(x_hbm.at[sidx.at[0]], rowsA, semA)

    def pair(jj, carry):
        j0 = 2 * jj
        pltpu.async_copy(x_hbm.at[sidx.at[j0 + 1]], rowsB, semB)
        pltpu.make_async_copy(x_hbm.at[sidx.at[j0]], rowsA, semA).wait()
        process(j0, rowsA, 0)

        @pl.when(j0 + 2 < GCH)
        def _():
            pltpu.async_copy(x_hbm.at[sidx.at[j0 + 2]], rowsA, semA)

        pltpu.make_async_copy(x_hbm.at[sidx.at[j0 + 1]], rowsB, semB).wait()
        process(j0 + 1, rowsB, 1)
        return carry

    lax.fori_loop(0, GCH // 2, pair, 0)
    plsc.subcore_barrier()
    pltpu.sync_copy(acc.at[pl.ds(s * HROWS, HROWS)],
                    out_hbm.at[pl.ds(c * HALF + s * HROWS, HROWS)])


@functools.partial(
    pl.kernel,
    out_type=jax.ShapeDtypeStruct((NW, SCH, SB, 16), jnp.float32),
    mesh=_MESH,
    scratch_types=[
        pltpu.VMEM((SCH, SB), jnp.int32),
        pltpu.VMEM((SCH, SB), jnp.int32),
        pltpu.VMEM((2, SB, D), jnp.float32),
        pltpu.VMEM((2, SB, D), jnp.float32),
        pltpu.VMEM((SB, 16), jnp.float32),
        pltpu.SemaphoreType.DMA,
        pltpu.SemaphoreType.DMA,
        pltpu.SemaphoreType.DMA,
        pltpu.SemaphoreType.DMA,
    ],
)
def _score_kernel(h_hbm, uix_hbm, vix_hbm, out_hbm, uix, vix, ub2, vb2, part,
                  semuA, semvA, semuB, semvB):
    c = lax.axis_index("c")
    s = lax.axis_index("s")
    wid = s * NC + c
    pltpu.sync_copy(uix_hbm.at[wid], uix)
    pltpu.sync_copy(vix_hbm.at[wid], vix)

    def compute(j, b):
        def edge(e, c2):
            acc = ub2[b, e, pl.ds(0, 16)] * vb2[b, e, pl.ds(0, 16)]
            for k in range(1, D // 16):
                acc = acc + (ub2[b, e, pl.ds(k * 16, 16)]
                             * vb2[b, e, pl.ds(k * 16, 16)])
            part[e] = acc
            return c2

        lax.fori_loop(0, SB, edge, 0)
        pltpu.sync_copy(part, out_hbm.at[wid, j])

    def fetch(j, b, su, sv):
        pltpu.async_copy(h_hbm.at[uix.at[j]], ub2.at[b], su)
        pltpu.async_copy(h_hbm.at[vix.at[j]], vb2.at[b], sv)

    def drain(j, b, su, sv):
        pltpu.make_async_copy(h_hbm.at[uix.at[j]], ub2.at[b], su).wait()
        pltpu.make_async_copy(h_hbm.at[vix.at[j]], vb2.at[b], sv).wait()

    fetch(0, 0, semuA, semvA)

    def pair(jj, carry):
        j0 = 2 * jj
        fetch(j0 + 1, 1, semuB, semvB)
        drain(j0, 0, semuA, semvA)
        compute(j0, 0)

        @pl.when(j0 + 2 < SCH)
        def _():
            fetch(j0 + 2, 0, semuA, semvA)

        drain(j0 + 1, 1, semuB, semvB)
        compute(j0 + 1, 1)
        return carry

    lax.fori_loop(0, SCH // 2, pair, 0)


RB = 1024


def _scale1_body(d_ref, emb_ref, out_ref):
    d = jnp.maximum(d_ref[...][:, :1], 1.0)
    out_ref[...] = emb_ref[...] * lax.rsqrt(d)


def _tc_scale1(deg, emb_p):
    return pl.pallas_call(
        _scale1_body,
        out_shape=jax.ShapeDtypeStruct((NPAD, D), jnp.float32),
        grid=(NPAD // RB,),
        in_specs=[
            pl.BlockSpec((RB, D), lambda i: (i, 0)),
            pl.BlockSpec((RB, D), lambda i: (i, 0)),
        ],
        out_specs=pl.BlockSpec((RB, D), lambda i: (i, 0)),
    )(deg, emb_p)


def _scale2_body(d_ref, p_ref, out_ref):
    d = jnp.maximum(d_ref[...][:, :1], 1.0)
    out_ref[...] = p_ref[...] / d


def _tc_scale2(deg, p):
    return pl.pallas_call(
        _scale2_body,
        out_shape=jax.ShapeDtypeStruct((NPAD, D), jnp.float32),
        grid=(NPAD // RB,),
        in_specs=[
            pl.BlockSpec((RB, D), lambda i: (i, 0)),
            pl.BlockSpec((RB, D), lambda i: (i, 0)),
        ],
        out_specs=pl.BlockSpec((RB, D), lambda i: (i, 0)),
    )(deg, p)


def _final_body(d_ref, p_ref, w_ref, b_ref, out_ref):
    d = jnp.maximum(d_ref[...][:, :1], 1.0)
    x = p_ref[...] * lax.rsqrt(d)
    out_ref[...] = (
        lax.dot_general(
            x, w_ref[...], (((1,), (1,)), ((), ())),
            preferred_element_type=jnp.float32,
            precision=lax.Precision.HIGHEST,
        )
        + b_ref[...]
    )


def _tc_final(deg, p, fc_w, fc_b2):
    return pl.pallas_call(
        _final_body,
        out_shape=jax.ShapeDtypeStruct((NPAD, D), jnp.float32),
        grid=(NPAD // RB,),
        in_specs=[
            pl.BlockSpec((RB, D), lambda i: (i, 0)),
            pl.BlockSpec((RB, D), lambda i: (i, 0)),
            pl.BlockSpec((D, D), lambda i: (0, 0)),
            pl.BlockSpec((1, D), lambda i: (0, 0)),
        ],
        out_specs=pl.BlockSpec((RB, D), lambda i: (i, 0)),
    )(deg, p, fc_w, fc_b2)


RB2 = 4000


def _rowsum_body(x_ref, out_ref):
    r16 = lax.broadcasted_iota(jnp.int32, (D, 8), 0) // 16
    cix = lax.broadcasted_iota(jnp.int32, (D, 8), 1)
    m = (r16 == cix).astype(jnp.float32)
    out_ref[...] = lax.dot_general(
        x_ref[...], m, (((1,), (0,)), ((), ())),
        preferred_element_type=jnp.float32,
        precision=lax.Precision.HIGHEST,
    )


def _tc_rowsum(scores16):
    rows = scores16.shape[0]
    return pl.pallas_call(
        _rowsum_body,
        out_shape=jax.ShapeDtypeStruct((rows, 8), jnp.float32),
        grid=(rows // RB2,),
        in_specs=[pl.BlockSpec((RB2, D), lambda i: (i, 0))],
        out_specs=pl.BlockSpec((RB2, 8), lambda i: (i, 0)),
    )(scores16)


def kernel(emb, fc_w, fc_b, g_edge_index, neg_edge_index):
    src3 = g_edge_index[0].reshape(NS, GCH, GB)
    dst3 = g_edge_index[1].reshape(NS, GCH, GB)
    emb_p = jnp.pad(emb, ((0, NPAD - N), (0, 0)))
    zeros_nd = jnp.zeros((NPAD, D), jnp.float32)
    ones_nd = jnp.ones((NPAD, D), jnp.float32)
    deg = _prop_kernel(ones_nd, src3, dst3, zeros_nd)
    t1 = _tc_scale1(deg, emb_p)
    p1 = _prop_kernel(t1, src3, dst3, zeros_nd)
    t3 = _tc_scale2(deg, p1)
    p2 = _prop_kernel(t3, src3, dst3, zeros_nd)
    h = _tc_final(deg, p2, fc_w, fc_b.reshape(1, D))
    u = jnp.concatenate([g_edge_index[0], neg_edge_index[0]]).reshape(NW, SCH, SB)
    v = jnp.concatenate([g_edge_index[1], neg_edge_index[1]]).reshape(NW, SCH, SB)
    scores16 = _score_kernel(h, u, v).reshape(2 * E * 16 // D, D)
    scores = _tc_rowsum(scores16).reshape(-1)
    return (scores[:E].reshape(E, 1), scores[E:].reshape(E, 1))


# speedup vs baseline: 3.4371x; 2.3436x over previous
import functools

import jax
import jax.numpy as jnp
from jax import lax
from jax.experimental import pallas as pl
from jax.experimental.pallas import tpu as pltpu
from jax.experimental.pallas import tpu_sc as plsc

N = 10000
NPAD = 10240
D = 128
E = 320000

NC = 2
NS = 16
NW = NC * NS

DEG_W = 16
HALF = NPAD // NC
HROWS = HALF // NS
ACC_R = HALF + 8
DUMP = HALF

EPS = E // NS
GB = 80
GCH = EPS // GB
FIFTHS = 5
GCF = GCH // FIFTHS
SCB = 16
NSUB = GB // SCB
SCCH = EPS // SCB

SB = 80
SCH = (2 * E) // NW // SB

_MESH = plsc.VectorSubcoreMesh(
    core_axis_name="c", subcore_axis_name="s", num_cores=NC, num_subcores=NS
)


RBD = 2000


def _dedup_body(x_ref, dm0_ref, dm1_ref, fp_ref):
    x = x_ref[...]
    col = lax.broadcasted_iota(jnp.int32, (RBD, SCB), 1)
    fp = col
    for p in range(SCB - 2, -1, -1):
        match = jnp.logical_and(x == x[:, p:p + 1], col > p)
        fp = jnp.where(match, p, fp)
    first = fp == col
    dm0_ref[...] = jnp.where(jnp.logical_and(first, x < HALF), x, DUMP)
    dm1_ref[...] = jnp.where(jnp.logical_and(first, x >= HALF), x - HALF, DUMP)
    hasdup = jnp.any(jnp.logical_not(first), axis=1, keepdims=True)
    fp_ref[...] = jnp.where(col == 0, jnp.where(hasdup, 0, -1), fp)


def _tc_dedup(dst16):
    rows = dst16.shape[0]
    oshape = jax.ShapeDtypeStruct((rows, SCB), jnp.int32)
    return pl.pallas_call(
        _dedup_body,
        out_shape=(oshape, oshape, oshape),
        grid=(rows // RBD,),
        in_specs=[pl.BlockSpec((RBD, SCB), lambda i: (i, 0))],
        out_specs=(
            pl.BlockSpec((RBD, SCB), lambda i: (i, 0)),
            pl.BlockSpec((RBD, SCB), lambda i: (i, 0)),
            pl.BlockSpec((RBD, SCB), lambda i: (i, 0)),
        ),
    )(dst16)


@functools.partial(
    pl.kernel,
    out_type=jax.ShapeDtypeStruct((NPAD, D), jnp.float32),
    mesh=_MESH,
    scratch_types=[
        pltpu.VMEM((GCF, GB), jnp.int32),
        pltpu.VMEM((GCF, GB), jnp.int32),
        pltpu.VMEM((GCF, GB), jnp.int32),
        pltpu.VMEM((2, NSUB, SCB), jnp.int32),
        pltpu.VMEM((GB, D), jnp.float32),
        pltpu.VMEM((GB, D), jnp.float32),
        pltpu.VMEM_SHARED((ACC_R, D), jnp.float32),
        pltpu.SemaphoreType.DMA,
        pltpu.SemaphoreType.DMA,
        pltpu.SemaphoreType.DMA,
    ],
)
def _prop_kernel(x_hbm, src3_hbm, dm0_hbm, dm1_hbm, fp3_hbm, zeros_hbm, out_hbm,
                 sidx, didx, fpq, dsx, rowsA, rowsB, acc, semA, semB, semS):
    c = lax.axis_index("c")
    s = lax.axis_index("s")
    pltpu.sync_copy(zeros_hbm.at[pl.ds(s * HROWS, HROWS)],
                    acc.at[pl.ds(s * HROWS, HROWS)])
    plsc.subcore_barrier()

    def process(j, rows, b):
        descs = []
        for m in range(NSUB):
            dsx[b, m] = didx[j, pl.ds(m * SCB, SCB)]
            fpv = fpq[j, pl.ds(m * SCB, SCB)]
            flag = fpv[0]

            @pl.when(flag >= 0)
            def _():
                for e in range(1, SCB):
                    fpe = fpv[e]

                    @pl.when(fpe < e)
                    def _():
                        for k in range(D // 16):
                            rows[m * SCB + fpe, pl.ds(k * 16, 16)] = (
                                rows[m * SCB + fpe, pl.ds(k * 16, 16)]
                                + rows[m * SCB + e, pl.ds(k * 16, 16)]
                            )

            descs.append(pltpu.async_copy(
                rows.at[pl.ds(m * SCB, SCB)],
                acc.at[dsx.at[b, m]], semS, add=True))
        for dd in descs:
            dd.wait()

    def slab(h, carry0):
        pltpu.sync_copy(src3_hbm.at[s, h], sidx)

        @pl.when(c == 0)
        def _():
            pltpu.sync_copy(dm0_hbm.at[s, h], didx)

        @pl.when(c == 1)
        def _():
            pltpu.sync_copy(dm1_hbm.at[s, h], didx)

        pltpu.sync_copy(fp3_hbm.at[s, h], fpq)
        pltpu.async_copy(x_hbm.at[sidx.at[0]], rowsA, semA)

        def pair(jj, carry):
            j0 = 2 * jj
            pltpu.async_copy(x_hbm.at[sidx.at[j0 + 1]], rowsB, semB)
            pltpu.make_async_copy(x_hbm.at[sidx.at[j0]], rowsA, semA).wait()
            process(j0, rowsA, 0)

            @pl.when(j0 + 2 < GCF)
            def _():
                pltpu.async_copy(x_hbm.at[sidx.at[j0 + 2]], rowsA, semA)

            pltpu.make_async_copy(x_hbm.at[sidx.at[j0 + 1]], rowsB, semB).wait()
            process(j0 + 1, rowsB, 1)
            return carry

        lax.fori_loop(0, GCF // 2, pair, 0)
        return carry0

    lax.fori_loop(0, FIFTHS, slab, 0)
    plsc.subcore_barrier()
    pltpu.sync_copy(acc.at[pl.ds(s * HROWS, HROWS)],
                    out_hbm.at[pl.ds(c * HALF + s * HROWS, HROWS)])


@functools.partial(
    pl.kernel,
    out_type=jax.ShapeDtypeStruct((NW, SCH, SB, 16), jnp.float32),
    mesh=_MESH,
    scratch_types=[
        pltpu.VMEM((SCH, SB), jnp.int32),
        pltpu.VMEM((SCH, SB), jnp.int32),
        pltpu.VMEM((2, SB, D), jnp.float32),
        pltpu.VMEM((2, SB, D), jnp.float32),
        pltpu.VMEM((SB, 16), jnp.float32),
        pltpu.SemaphoreType.DMA,
        pltpu.SemaphoreType.DMA,
        pltpu.SemaphoreType.DMA,
        pltpu.SemaphoreType.DMA,
    ],
)
def _score_kernel(h_hbm, uix_hbm, vix_hbm, out_hbm, uix, vix, ub2, vb2, part,
                  semuA, semvA, semuB, semvB):
    c = lax.axis_index("c")
    s = lax.axis_index("s")
    wid = s * NC + c
    pltpu.sync_copy(uix_hbm.at[wid], uix)
    pltpu.sync_copy(vix_hbm.at[wid], vix)

    def compute(j, b):
        def edge(e, c2):
            acc = ub2[b, e, pl.ds(0, 16)] * vb2[b, e, pl.ds(0, 16)]
            for k in range(1, D // 16):
                acc = acc + (ub2[b, e, pl.ds(k * 16, 16)]
                             * vb2[b, e, pl.ds(k * 16, 16)])
            part[e] = acc
            return c2

        lax.fori_loop(0, SB, edge, 0)
        pltpu.sync_copy(part, out_hbm.at[wid, j])

    def fetch(j, b, su, sv):
        pltpu.async_copy(h_hbm.at[uix.at[j]], ub2.at[b], su)
        pltpu.async_copy(h_hbm.at[vix.at[j]], vb2.at[b], sv)

    def drain(j, b, su, sv):
        pltpu.make_async_copy(h_hbm.at[uix.at[j]], ub2.at[b], su).wait()
        pltpu.make_async_copy(h_hbm.at[vix.at[j]], vb2.at[b], sv).wait()

    fetch(0, 0, semuA, semvA)

    def pair(jj, carry):
        j0 = 2 * jj
        fetch(j0 + 1, 1, semuB, semvB)
        drain(j0, 0, semuA, semvA)
        compute(j0, 0)

        @pl.when(j0 + 2 < SCH)
        def _():
            fetch(j0 + 2, 0, semuA, semvA)

        drain(j0 + 1, 1, semuB, semvB)
        compute(j0 + 1, 1)
        return carry

    lax.fori_loop(0, SCH // 2, pair, 0)


RB = 1024


def _scale1_body(d_ref, emb_ref, out_ref):
    d = jnp.maximum(d_ref[...][:, :1], 1.0)
    out_ref[...] = emb_ref[...] * lax.rsqrt(d)


def _tc_scale1(deg, emb_p):
    return pl.pallas_call(
        _scale1_body,
        out_shape=jax.ShapeDtypeStruct((NPAD, D), jnp.float32),
        grid=(NPAD // RB,),
        in_specs=[
            pl.BlockSpec((RB, D), lambda i: (i, 0)),
            pl.BlockSpec((RB, D), lambda i: (i, 0)),
        ],
        out_specs=pl.BlockSpec((RB, D), lambda i: (i, 0)),
    )(deg, emb_p)


def _scale2_body(d_ref, p_ref, out_ref):
    d = jnp.maximum(d_ref[...][:, :1], 1.0)
    out_ref[...] = p_ref[...] / d


def _tc_scale2(deg, p):
    return pl.pallas_call(
        _scale2_body,
        out_shape=jax.ShapeDtypeStruct((NPAD, D), jnp.float32),
        grid=(NPAD // RB,),
        in_specs=[
            pl.BlockSpec((RB, D), lambda i: (i, 0)),
            pl.BlockSpec((RB, D), lambda i: (i, 0)),
        ],
        out_specs=pl.BlockSpec((RB, D), lambda i: (i, 0)),
    )(deg, p)


def _final_body(d_ref, p_ref, w_ref, b_ref, out_ref):
    d = jnp.maximum(d_ref[...][:, :1], 1.0)
    x = p_ref[...] * lax.rsqrt(d)
    out_ref[...] = (
        lax.dot_general(
            x, w_ref[...], (((1,), (1,)), ((), ())),
            preferred_element_type=jnp.float32,
            precision=lax.Precision.HIGHEST,
        )
        + b_ref[...]
    )


def _tc_final(deg, p, fc_w, fc_b2):
    return pl.pallas_call(
        _final_body,
        out_shape=jax.ShapeDtypeStruct((NPAD, D), jnp.float32),
        grid=(NPAD // RB,),
        in_specs=[
            pl.BlockSpec((RB, D), lambda i: (i, 0)),
            pl.BlockSpec((RB, D), lambda i: (i, 0)),
            pl.BlockSpec((D, D), lambda i: (0, 0)),
            pl.BlockSpec((1, D), lambda i: (0, 0)),
        ],
        out_specs=pl.BlockSpec((RB, D), lambda i: (i, 0)),
    )(deg, p, fc_w, fc_b2)


RB2 = 4000


def _rowsum_body(x_ref, out_ref):
    r16 = lax.broadcasted_iota(jnp.int32, (D, 8), 0) // 16
    cix = lax.broadcasted_iota(jnp.int32, (D, 8), 1)
    m = (r16 == cix).astype(jnp.float32)
    out_ref[...] = lax.dot_general(
        x_ref[...], m, (((1,), (0,)), ((), ())),
        preferred_element_type=jnp.float32,
        precision=lax.Precision.HIGHEST,
    )


def _tc_rowsum(scores16):
    rows = scores16.shape[0]
    return pl.pallas_call(
        _rowsum_body,
        out_shape=jax.ShapeDtypeStruct((rows, 8), jnp.float32),
        grid=(rows // RB2,),
        in_specs=[pl.BlockSpec((RB2, D), lambda i: (i, 0))],
        out_specs=pl.BlockSpec((RB2, 8), lambda i: (i, 0)),
    )(scores16)


def kernel(emb, fc_w, fc_b, g_edge_index, neg_edge_index):
    src3 = g_edge_index[0].reshape(NS, FIFTHS, GCF, GB)
    dst16 = g_edge_index[1].reshape(E // SCB, SCB)
    emb_p = jnp.pad(emb, ((0, NPAD - N), (0, 0)))
    zeros_nd = jnp.zeros((NPAD, D), jnp.float32)
    ones_nd = jnp.ones((NPAD, D), jnp.float32)
    dm0, dm1, fp3 = _tc_dedup(dst16)
    dm0 = dm0.reshape(NS, FIFTHS, GCF, GB)
    dm1 = dm1.reshape(NS, FIFTHS, GCF, GB)
    fp3 = fp3.reshape(NS, FIFTHS, GCF, GB)
    deg = _prop_kernel(ones_nd, src3, dm0, dm1, fp3, zeros_nd)
    t1 = _tc_scale1(deg, emb_p)
    p1 = _prop_kernel(t1, src3, dm0, dm1, fp3, zeros_nd)
    t3 = _tc_scale2(deg, p1)
    p2 = _prop_kernel(t3, src3, dm0, dm1, fp3, zeros_nd)
    h = _tc_final(deg, p2, fc_w, fc_b.reshape(1, D))
    u = jnp.concatenate([g_edge_index[0], neg_edge_index[0]]).reshape(NW, SCH, SB)
    v = jnp.concatenate([g_edge_index[1], neg_edge_index[1]]).reshape(NW, SCH, SB)
    scores16 = _score_kernel(h, u, v).reshape(2 * E * 16 // D, D)
    scores = _tc_rowsum(scores16).reshape(-1)
    return (scores[:E].reshape(E, 1), scores[E:].reshape(E, 1))
